# baseline (speedup 1.0000x reference)
import functools

import jax
import jax.numpy as jnp
from jax import lax
from jax.experimental import pallas as pl
from jax.experimental.pallas import tpu as pltpu
from jax.experimental.pallas import tpu_sc as plsc

N = 2048
D = 768
T = 8
NC, NS, L = 2, 16, 16
NW = NC * NS
CHUNK = N // NW
BM = 128
NB = N // BM

_MESH = plsc.VectorSubcoreMesh(
    core_axis_name="c", subcore_axis_name="s", num_cores=NC, num_subcores=NS)


@functools.partial(
    pl.kernel,
    out_type=(
        jax.ShapeDtypeStruct((N, D), jnp.float32),
        jax.ShapeDtypeStruct((N,), jnp.int32),
        jax.ShapeDtypeStruct((L,), jnp.int32),
    ),
    mesh=_MESH,
    scratch_types=[
        pltpu.VMEM((N,), jnp.int32),
        pltpu.VMEM((CHUNK,), jnp.int32),
        pltpu.VMEM((CHUNK, D), jnp.float32),
        pltpu.VMEM((L,), jnp.int32),
        pltpu.SemaphoreType.DMA,
    ],
    compiler_params=pltpu.CompilerParams(needs_layout_passes=False),
)
def _route_kernel(x_hbm, xt_hbm, xs_hbm, pos_hbm, off_hbm,
                  ty_v, pos_v, rows_v, off_v, sem):
    w = lax.axis_index("s") * NC + lax.axis_index("c")
    base = w * CHUNK
    iota16 = lax.broadcasted_iota(jnp.int32, (L,), 0)

    pltpu.sync_copy(xt_hbm, ty_v)

    one = jnp.full((L,), 1, jnp.int32)
    zero = jnp.zeros((L,), jnp.int32)
    tot_l = [zero for _ in range(T)]
    mine_l = [zero for _ in range(T)]
    for ci in range(NW):
        snap = jnp.full((L,), ci == w)
        for t in range(T):
            mine_l[t] = jnp.where(snap, tot_l[t], mine_l[t])
        for sub in range(CHUNK // L):
            tyc = ty_v[pl.ds(ci * CHUNK + sub * L, L)]
            for t in range(T):
                tot_l[t] = tot_l[t] + jnp.where(
                    tyc == jnp.full((L,), t), one, zero)
    total = jnp.zeros((L,), jnp.int32)
    mine = jnp.zeros((L,), jnp.int32)
    for t in range(T):
        st = jnp.sum(tot_l[t])
        sm = jnp.sum(mine_l[t])
        total = jnp.where(iota16 == t, jnp.full((L,), st), total)
        mine = jnp.where(iota16 == t, jnp.full((L,), sm), mine)
    cs = plsc.cumsum(total)
    excl = cs - total
    base_vec = excl + mine

    for ci in range(CHUNK // L):
        tyc = ty_v[pl.ds(base + ci * L, L)]
        posc = jnp.zeros((L,), jnp.int32)
        for t in range(T):
            m = tyc == t
            cm = plsc.cumsum(m.astype(jnp.int32))
            bt = jnp.sum(jnp.where(iota16 == t, base_vec, 0))
            posc = jnp.where(m, jnp.full((L,), bt) + cm - 1, posc)
            cnt = jnp.max(cm)
            base_vec = jnp.where(iota16 == t, base_vec + jnp.full((L,), cnt),
                                 base_vec)
        posc = jnp.minimum(jnp.maximum(posc, jnp.zeros((L,), jnp.int32)),
                           jnp.full((L,), N - 1))
        pos_v[pl.ds(ci * L, L)] = posc

    pltpu.sync_copy(pos_v, pos_hbm.at[pl.ds(base, CHUNK)])
    pltpu.sync_copy(x_hbm.at[pl.ds(base, CHUNK)], rows_v)
    pltpu.async_copy(rows_v, xs_hbm.at[pos_v], sem).wait()

    @pl.when(w == 0)
    def _offsets():
        off_v[...] = excl
        pltpu.sync_copy(off_v, off_hbm)


def _gemm_body(off_ref, x_ref, w_ref, b_ref, o_ref):
    t = pl.program_id(0)
    ri = lax.broadcasted_iota(jnp.int32, (BM, 1), 0)
    seg_s = off_ref[t]
    seg_e = off_ref[t + 1]
    for b in range(NB):
        lo = b * BM
        s = seg_s - lo
        e = seg_e - lo

        @pl.when((e > s) & (e > 0) & (s < BM))
        def _acc():
            xb = x_ref[pl.ds(lo, BM), :].astype(jnp.bfloat16)
            prod = jnp.dot(xb, w_ref[0],
                           preferred_element_type=jnp.float32)
            prod = prod + b_ref[t][None, :]
            m = (ri >= s) & (ri < e)
            o_ref[pl.ds(lo, BM), :] = jnp.where(m, prod,
                                                o_ref[pl.ds(lo, BM), :])


def _grouped_gemm(offs, xs, W, b):
    grid_spec = pltpu.PrefetchScalarGridSpec(
        num_scalar_prefetch=1,
        grid=(T,),
        in_specs=[
            pl.BlockSpec((N, D), lambda g, off: (0, 0)),
            pl.BlockSpec((1, D, D), lambda g, off: (g, 0, 0)),
            pl.BlockSpec((T, D), lambda g, off: (0, 0)),
        ],
        out_specs=pl.BlockSpec((N, D), lambda g, off: (0, 0)),
    )
    return pl.pallas_call(
        _gemm_body,
        grid_spec=grid_spec,
        out_shape=jax.ShapeDtypeStruct((N, D), jnp.float32),
    )(offs, xs, W, b)


@functools.partial(
    pl.kernel,
    out_type=jax.ShapeDtypeStruct((N, D), jnp.float32),
    mesh=_MESH,
    scratch_types=[
        pltpu.VMEM((CHUNK,), jnp.int32),
        pltpu.VMEM((CHUNK, D), jnp.float32),
        pltpu.SemaphoreType.DMA,
    ],
    compiler_params=pltpu.CompilerParams(needs_layout_passes=False),
)
def _unsort_kernel(y_hbm, pos_hbm, out_hbm, pos_v, rows_v, sem):
    w = lax.axis_index("s") * NC + lax.axis_index("c")
    base = w * CHUNK
    pltpu.sync_copy(pos_hbm.at[pl.ds(base, CHUNK)], pos_v)
    pltpu.async_copy(y_hbm.at[pos_v], rows_v, sem).wait()
    pltpu.sync_copy(rows_v, out_hbm.at[pl.ds(base, CHUNK)])


@jax.jit
def kernel(x, x_type, W, b):
    xt = x_type.astype(jnp.int32)
    xs, pos, offs = _route_kernel(x, xt)
    y = _grouped_gemm(offs, xs, W.astype(jnp.bfloat16), b)
    return _unsort_kernel(y, pos)


# speedup vs baseline: 1.0601x; 1.0601x over previous
import functools

import jax
import jax.numpy as jnp
from jax import lax
from jax.experimental import pallas as pl
from jax.experimental.pallas import tpu as pltpu
from jax.experimental.pallas import tpu_sc as plsc

N = 2048
D = 768
T = 8
NC, NS, L = 2, 16, 16
NW = NC * NS
CHUNK = N // NW
BM = 256
NB = N // BM

_MESH = plsc.VectorSubcoreMesh(
    core_axis_name="c", subcore_axis_name="s", num_cores=NC, num_subcores=NS)


@functools.partial(
    pl.kernel,
    out_type=(
        jax.ShapeDtypeStruct((N, D), jnp.float32),
        jax.ShapeDtypeStruct((N,), jnp.int32),
        jax.ShapeDtypeStruct((L,), jnp.int32),
    ),
    mesh=_MESH,
    scratch_types=[
        pltpu.VMEM((N,), jnp.int32),
        pltpu.VMEM((CHUNK,), jnp.int32),
        pltpu.VMEM((CHUNK, D), jnp.float32),
        pltpu.VMEM((L,), jnp.int32),
        pltpu.SemaphoreType.DMA,
    ],
    compiler_params=pltpu.CompilerParams(needs_layout_passes=False),
)
def _route_kernel(x_hbm, xt_hbm, xs_hbm, pos_hbm, off_hbm,
                  ty_v, pos_v, rows_v, off_v, sem):
    w = lax.axis_index("s") * NC + lax.axis_index("c")
    base = w * CHUNK
    iota16 = lax.broadcasted_iota(jnp.int32, (L,), 0)

    rows_cp = pltpu.async_copy(x_hbm.at[pl.ds(base, CHUNK)], rows_v, sem)
    pltpu.sync_copy(xt_hbm, ty_v)

    one = jnp.full((L,), 1, jnp.int32)
    zero = jnp.zeros((L,), jnp.int32)
    tot_l = [zero for _ in range(T)]
    mine_l = [zero for _ in range(T)]
    for ci in range(NW):
        snap = jnp.full((L,), ci == w)
        for t in range(T):
            mine_l[t] = jnp.where(snap, tot_l[t], mine_l[t])
        for sub in range(CHUNK // L):
            tyc = ty_v[pl.ds(ci * CHUNK + sub * L, L)]
            for t in range(T):
                tot_l[t] = tot_l[t] + jnp.where(
                    tyc == jnp.full((L,), t), one, zero)
    total = jnp.zeros((L,), jnp.int32)
    mine = jnp.zeros((L,), jnp.int32)
    for t in range(T):
        st = jnp.sum(tot_l[t])
        sm = jnp.sum(mine_l[t])
        total = jnp.where(iota16 == t, jnp.full((L,), st), total)
        mine = jnp.where(iota16 == t, jnp.full((L,), sm), mine)
    cs = plsc.cumsum(total)
    excl = cs - total
    base_vec = excl + mine

    for ci in range(CHUNK // L):
        tyc = ty_v[pl.ds(base + ci * L, L)]
        posc = jnp.zeros((L,), jnp.int32)
        for t in range(T):
            m = tyc == t
            cm = plsc.cumsum(m.astype(jnp.int32))
            bt = jnp.sum(jnp.where(iota16 == t, base_vec, 0))
            posc = jnp.where(m, jnp.full((L,), bt) + cm - 1, posc)
            cnt = jnp.max(cm)
            base_vec = jnp.where(iota16 == t, base_vec + jnp.full((L,), cnt),
                                 base_vec)
        posc = jnp.minimum(jnp.maximum(posc, jnp.zeros((L,), jnp.int32)),
                           jnp.full((L,), N - 1))
        pos_v[pl.ds(ci * L, L)] = posc

    pltpu.sync_copy(pos_v, pos_hbm.at[pl.ds(base, CHUNK)])
    rows_cp.wait()
    pltpu.async_copy(rows_v, xs_hbm.at[pos_v], sem).wait()

    @pl.when(w == 0)
    def _offsets():
        off_v[...] = excl
        pltpu.sync_copy(off_v, off_hbm)


def _gemm_body(off_ref, x_ref, w_ref, b_ref, o_ref):
    t = pl.program_id(0)
    ri = lax.broadcasted_iota(jnp.int32, (BM, 1), 0)
    seg_s = off_ref[t]
    seg_e = off_ref[t + 1]
    for b in range(NB):
        lo = b * BM
        s = seg_s - lo
        e = seg_e - lo

        @pl.when((e > s) & (e > 0) & (s < BM))
        def _acc():
            prod = jnp.dot(x_ref[pl.ds(lo, BM), :], w_ref[0],
                           preferred_element_type=jnp.float32)
            prod = prod + b_ref[t][None, :]
            m = (ri >= s) & (ri < e)
            o_ref[pl.ds(lo, BM), :] = jnp.where(m, prod,
                                                o_ref[pl.ds(lo, BM), :])


def _grouped_gemm(offs, xs, W, b):
    grid_spec = pltpu.PrefetchScalarGridSpec(
        num_scalar_prefetch=1,
        grid=(T,),
        in_specs=[
            pl.BlockSpec((N, D), lambda g, off: (0, 0)),
            pl.BlockSpec((1, D, D), lambda g, off: (g, 0, 0)),
            pl.BlockSpec((T, D), lambda g, off: (0, 0)),
        ],
        out_specs=pl.BlockSpec((N, D), lambda g, off: (0, 0)),
    )
    return pl.pallas_call(
        _gemm_body,
        grid_spec=grid_spec,
        out_shape=jax.ShapeDtypeStruct((N, D), jnp.float32),
    )(offs, xs, W, b)


@functools.partial(
    pl.kernel,
    out_type=jax.ShapeDtypeStruct((N, D), jnp.float32),
    mesh=_MESH,
    scratch_types=[
        pltpu.VMEM((CHUNK // 2,), jnp.int32),
        pltpu.VMEM((CHUNK // 2,), jnp.int32),
        pltpu.VMEM((CHUNK // 2, D), jnp.float32),
        pltpu.VMEM((CHUNK // 2, D), jnp.float32),
        pltpu.SemaphoreType.DMA,
        pltpu.SemaphoreType.DMA,
    ],
    compiler_params=pltpu.CompilerParams(needs_layout_passes=False),
)
def _unsort_kernel(y_hbm, pos_hbm, out_hbm, pos_a, pos_b, rows_a, rows_b,
                   sem_a, sem_b):
    w = lax.axis_index("s") * NC + lax.axis_index("c")
    base = w * CHUNK
    half = CHUNK // 2
    pltpu.sync_copy(pos_hbm.at[pl.ds(base, half)], pos_a)
    ga = pltpu.async_copy(y_hbm.at[pos_a], rows_a, sem_a)
    pltpu.sync_copy(pos_hbm.at[pl.ds(base + half, half)], pos_b)
    gb = pltpu.async_copy(y_hbm.at[pos_b], rows_b, sem_b)
    ga.wait()
    pltpu.sync_copy(rows_a, out_hbm.at[pl.ds(base, half)])
    gb.wait()
    pltpu.sync_copy(rows_b, out_hbm.at[pl.ds(base + half, half)])


@jax.jit
def kernel(x, x_type, W, b):
    xt = x_type.astype(jnp.int32)
    xs, pos, offs = _route_kernel(x, xt)
    y = _grouped_gemm(offs, xs, W.astype(jnp.bfloat16), b)
    return _unsort_kernel(y, pos)
